# baseline (speedup 1.0000x reference)
import jax
import jax.numpy as jnp
from jax.experimental import pallas as pl

H, W = 50, 37
HP, WP = H + 2, W + 2
NFLAT = HP * WP
NC = 1948
CIN = 512
COUT = 512


def _rpn_kernel(x_ref, wt_ref, bc_ref, wcl_ref, bcl_ref, cls_ref, loc_ref):
    x = x_ref[0]
    xcat = jnp.concatenate(
        [x[:, (t // 3) * WP + (t % 3):(t // 3) * WP + (t % 3) + NC]
         for t in range(9)], axis=0)
    acc = jnp.dot(wt_ref[...], xcat, preferred_element_type=jnp.float32)
    h = jnp.maximum(acc + bc_ref[:, :1], 0.0)
    s = jnp.dot(wcl_ref[...], h,
                preferred_element_type=jnp.float32) + bcl_ref[:, :1]
    a = s[0:9]
    b = s[9:18]
    m = jnp.maximum(a, b)
    ea = jnp.exp(a - m)
    eb = jnp.exp(b - m)
    d = ea + eb
    cls_ref[0, :, :NC] = jnp.concatenate([ea / d, eb / d], axis=0)
    loc_ref[0, :, :NC] = s[18:54]


def kernel(feats, gt_boxes, im_info, W_conv, b_conv, W_cls, b_cls, W_loc, b_loc):
    B = feats.shape[0]
    xp = jnp.pad(feats, ((0, 0), (0, 0), (1, 1), (1, 1))).astype(jnp.bfloat16)
    xflat = xp.reshape(B, CIN, NFLAT)
    wbf = jax.lax.optimization_barrier(W_conv.astype(jnp.bfloat16))
    wt = jnp.transpose(wbf, (0, 2, 3, 1)).reshape(COUT, 9 * CIN)
    wcl = jnp.concatenate([W_cls[:, :, 0, 0], W_loc[:, :, 0, 0]], axis=0)
    bcl = jnp.concatenate([b_cls, b_loc])[:, None]
    bc = b_conv[:, None]

    cls_flat, loc_flat = pl.pallas_call(
        _rpn_kernel,
        grid=(B,),
        in_specs=[
            pl.BlockSpec((1, CIN, NFLAT), lambda i: (i, 0, 0)),
            pl.BlockSpec((COUT, 9 * CIN), lambda i: (0, 0)),
            pl.BlockSpec((COUT, 1), lambda i: (0, 0)),
            pl.BlockSpec((54, CIN), lambda i: (0, 0)),
            pl.BlockSpec((54, 1), lambda i: (0, 0)),
        ],
        out_specs=[
            pl.BlockSpec((1, 18, H * WP), lambda i: (i, 0, 0)),
            pl.BlockSpec((1, 36, H * WP), lambda i: (i, 0, 0)),
        ],
        out_shape=[
            jax.ShapeDtypeStruct((B, 18, H * WP), jnp.float32),
            jax.ShapeDtypeStruct((B, 36, H * WP), jnp.float32),
        ],
    )(xflat, wt, bc, wcl, bcl)

    cls = cls_flat.reshape(B, 18, H, WP)[:, :, :, :W]
    loc = loc_flat.reshape(B, 36, H, WP)[:, :, :, :W]
    return (cls, loc)


# speedup vs baseline: 1.0103x; 1.0103x over previous
import jax
import jax.numpy as jnp
from jax.experimental import pallas as pl

H, W = 50, 37
HP, WP = H + 2, W + 2
NFLAT = HP * WP
NC = 1948
CIN = 512
COUT = 512


G = 2


def _rpn_kernel(x_ref, wt_ref, bc_ref, wcl_ref, bcl_ref, cls_ref, loc_ref):
    for b in range(G):
        x = x_ref[b]
        acc = jnp.zeros((COUT, NC), jnp.float32)
        for t in range(9):
            dy, dx = t // 3, t % 3
            off = dy * WP + dx
            acc = acc + jnp.dot(wt_ref[t], x[:, off:off + NC],
                                preferred_element_type=jnp.float32)
        h = jnp.maximum(acc + bc_ref[:, :1], 0.0)
        s = jnp.dot(wcl_ref[...], h,
                    preferred_element_type=jnp.float32) + bcl_ref[:, :1]
        a = s[0:9]
        b2 = s[9:18]
        m = jnp.maximum(a, b2)
        ea = jnp.exp(a - m)
        eb = jnp.exp(b2 - m)
        d = ea + eb
        cls_ref[b, :, :NC] = jnp.concatenate([ea / d, eb / d], axis=0)
        loc_ref[b, :, :NC] = s[18:54]


def kernel(feats, gt_boxes, im_info, W_conv, b_conv, W_cls, b_cls, W_loc, b_loc):
    B = feats.shape[0]
    xp = jnp.pad(feats, ((0, 0), (0, 0), (1, 1), (1, 1))).astype(jnp.bfloat16)
    xflat = xp.reshape(B, CIN, NFLAT)
    wbf = jax.lax.optimization_barrier(W_conv.astype(jnp.bfloat16))
    wt = jnp.transpose(wbf, (2, 3, 0, 1)).reshape(9, COUT, CIN)
    wcl = jnp.concatenate([W_cls[:, :, 0, 0], W_loc[:, :, 0, 0]], axis=0)
    bcl = jnp.concatenate([b_cls, b_loc])[:, None]
    bc = b_conv[:, None]

    cls_flat, loc_flat = pl.pallas_call(
        _rpn_kernel,
        grid=(B // G,),
        in_specs=[
            pl.BlockSpec((G, CIN, NFLAT), lambda i: (i, 0, 0)),
            pl.BlockSpec((9, COUT, CIN), lambda i: (0, 0, 0)),
            pl.BlockSpec((COUT, 1), lambda i: (0, 0)),
            pl.BlockSpec((54, CIN), lambda i: (0, 0)),
            pl.BlockSpec((54, 1), lambda i: (0, 0)),
        ],
        out_specs=[
            pl.BlockSpec((G, 18, H * WP), lambda i: (i, 0, 0)),
            pl.BlockSpec((G, 36, H * WP), lambda i: (i, 0, 0)),
        ],
        out_shape=[
            jax.ShapeDtypeStruct((B, 18, H * WP), jnp.float32),
            jax.ShapeDtypeStruct((B, 36, H * WP), jnp.float32),
        ],
    )(xflat, wt, bc, wcl, bcl)

    cls = cls_flat.reshape(B, 18, H, WP)[:, :, :, :W]
    loc = loc_flat.reshape(B, 36, H, WP)[:, :, :, :W]
    return (cls, loc)


# speedup vs baseline: 1.0206x; 1.0102x over previous
import jax
import jax.numpy as jnp
from jax.experimental import pallas as pl

H, W = 50, 37
HP, WP = H + 2, W + 2
NFLAT = HP * WP
NC = 1948
CIN = 512
COUT = 512


def _rpn_kernel(x_ref, wt_ref, bc_ref, wcl_ref, bcl_ref, cls_ref, loc_ref):
    x = x_ref[0]
    acc = jnp.zeros((NC, COUT), jnp.float32)
    for t in range(9):
        dy, dx = t // 3, t % 3
        off = dy * WP + dx
        acc = acc + jnp.dot(x[off:off + NC, :], wt_ref[t],
                            preferred_element_type=jnp.float32)
    h = jnp.maximum(acc + bc_ref[:1, :], 0.0)
    s_t = jnp.dot(h, wcl_ref[...],
                  preferred_element_type=jnp.float32) + bcl_ref[:1, :]
    s = jnp.transpose(s_t)
    a = s[0:9]
    b = s[9:18]
    m = jnp.maximum(a, b)
    ea = jnp.exp(a - m)
    eb = jnp.exp(b - m)
    d = ea + eb
    cls_ref[0, :, :NC] = jnp.concatenate([ea / d, eb / d], axis=0)
    loc_ref[0, :, :NC] = s[18:54]


def kernel(feats, gt_boxes, im_info, W_conv, b_conv, W_cls, b_cls, W_loc, b_loc):
    B = feats.shape[0]
    xp = jnp.pad(feats, ((0, 0), (0, 0), (1, 1), (1, 1)))
    xt = jnp.transpose(xp.reshape(B, CIN, NFLAT), (0, 2, 1)).astype(jnp.bfloat16)
    wbf = jax.lax.optimization_barrier(W_conv.astype(jnp.bfloat16))
    wt = jnp.transpose(wbf, (2, 3, 1, 0)).reshape(9, CIN, COUT)
    wcl = jnp.concatenate([W_cls[:, :, 0, 0], W_loc[:, :, 0, 0]], axis=0)
    wclt = jnp.transpose(wcl)
    bcl = jnp.concatenate([b_cls, b_loc])[None, :]
    bc = b_conv[None, :]

    cls_flat, loc_flat = pl.pallas_call(
        _rpn_kernel,
        grid=(B,),
        in_specs=[
            pl.BlockSpec((1, NFLAT, CIN), lambda i: (i, 0, 0)),
            pl.BlockSpec((9, CIN, COUT), lambda i: (0, 0, 0)),
            pl.BlockSpec((1, COUT), lambda i: (0, 0)),
            pl.BlockSpec((CIN, 54), lambda i: (0, 0)),
            pl.BlockSpec((1, 54), lambda i: (0, 0)),
        ],
        out_specs=[
            pl.BlockSpec((1, 18, H * WP), lambda i: (i, 0, 0)),
            pl.BlockSpec((1, 36, H * WP), lambda i: (i, 0, 0)),
        ],
        out_shape=[
            jax.ShapeDtypeStruct((B, 18, H * WP), jnp.float32),
            jax.ShapeDtypeStruct((B, 36, H * WP), jnp.float32),
        ],
    )(xt, wt, bc, wclt, bcl)

    cls = cls_flat.reshape(B, 18, H, WP)[:, :, :, :W]
    loc = loc_flat.reshape(B, 36, H, WP)[:, :, :, :W]
    return (cls, loc)
